# baseline (device time: 142227 ns/iter reference)
import jax
import jax.numpy as jnp
from jax import lax
from jax.experimental import pallas as pl
from jax.experimental.pallas import tpu as pltpu

N_DEV = 16
M_BLK = 512
K_BLK = 512
N_HALF = 2048


def kernel(x, w_mat):
    m_total, k_shard = x.shape
    k_total, n = w_mat.shape
    assert k_shard == K_BLK and m_total == N_DEV * M_BLK
    n_halves = n // N_HALF
    n_steps = N_DEV * n_halves

    def body(x_ref, w_ref, out_ref, staging, x_send, a2a_buf, w_buf,
             send_sems, recv_sems, st_sems, w_sems):
        me = lax.axis_index("i")

        def w_copy(step, slot):
            d, h = divmod(step, n_halves)
            s = lax.rem(me + (N_DEV - d), N_DEV)
            return pltpu.make_async_copy(
                w_ref.at[pl.ds(s * K_BLK, K_BLK), pl.ds(h * N_HALF, N_HALF)],
                w_buf.at[slot], w_sems.at[slot])

        w_copy(0, 0).start()

        barrier_sem = pltpu.get_barrier_semaphore()
        for d in range(1, N_DEV):
            t = lax.rem(me + d, N_DEV)
            pl.semaphore_signal(
                barrier_sem, inc=1,
                device_id=(t,), device_id_type=pl.DeviceIdType.MESH)
        pl.semaphore_wait(barrier_sem, N_DEV - 1)

        def stage_in(j, slot):
            t = lax.rem(me + j, N_DEV)
            return pltpu.make_async_copy(
                x_ref.at[pl.ds(t * M_BLK, M_BLK), :],
                staging.at[slot], st_sems.at[slot])

        seq = sorted(range(1, N_DEV), key=lambda d: abs(8 - d)) + [0]
        n_stage = 4
        for k in range(n_stage - 1):
            stage_in(seq[k], k).start()
        rdmas = []
        for idx, j in enumerate(seq):
            slot = idx % n_stage
            if idx + n_stage - 1 < len(seq):
                stage_in(seq[idx + n_stage - 1], (idx + n_stage - 1) % n_stage).start()
            stage_in(j, slot).wait()
            x_send[j] = staging[slot].astype(jnp.bfloat16)
            if j != 0:
                t = lax.rem(me + j, N_DEV)
                rdma = pltpu.make_async_remote_copy(
                    src_ref=x_send.at[j],
                    dst_ref=a2a_buf.at[j],
                    send_sem=send_sems.at[j],
                    recv_sem=recv_sems.at[j],
                    device_id=(t,),
                    device_id_type=pl.DeviceIdType.MESH,
                )
                rdma.start()
                rdmas.append((j, rdma))
        rdma_by_offset = dict(rdmas)

        for d in range(N_DEV):
            for h in range(n_halves):
                step = d * n_halves + h
                if step + 1 < n_steps:
                    w_copy(step + 1, (step + 1) % 2).start()
                w_copy(step, step % 2).wait()
                if h == 0 and d > 0:
                    rdma_by_offset[d].wait_recv()
                lhs = (x_send[0] if d == 0 else a2a_buf[d]).astype(jnp.float32)
                prod = lax.dot_general(
                    lhs, w_buf[step % 2],
                    (((1,), (0,)), ((), ())),
                    preferred_element_type=jnp.float32,
                )
                cols = pl.ds(h * N_HALF, N_HALF)
                if d == 0:
                    out_ref[:, cols] = prod
                else:
                    out_ref[:, cols] += prod

        out_ref[...] = jnp.maximum(out_ref[...], 0.0)

        for _, r in rdmas:
            r.wait_send()

    return pl.pallas_call(
        body,
        out_shape=jax.ShapeDtypeStruct((M_BLK, n), jnp.float32),
        in_specs=[
            pl.BlockSpec(memory_space=pl.ANY),
            pl.BlockSpec(memory_space=pl.ANY),
        ],
        out_specs=pl.BlockSpec(memory_space=pltpu.MemorySpace.VMEM),
        scratch_shapes=[
            pltpu.VMEM((4, M_BLK, K_BLK), jnp.float32),
            pltpu.VMEM((N_DEV, M_BLK, K_BLK), jnp.bfloat16),
            pltpu.VMEM((N_DEV, M_BLK, K_BLK), jnp.bfloat16),
            pltpu.VMEM((2, K_BLK, N_HALF), jnp.float32),
            pltpu.SemaphoreType.DMA((N_DEV,)),
            pltpu.SemaphoreType.DMA((N_DEV,)),
            pltpu.SemaphoreType.DMA((4,)),
            pltpu.SemaphoreType.DMA((2,)),
        ],
        compiler_params=pltpu.CompilerParams(collective_id=0),
    )(x, w_mat)


# device time: 116017 ns/iter; 1.2259x vs baseline; 1.2259x over previous
import jax
import jax.numpy as jnp
from jax import lax
from jax.experimental import pallas as pl
from jax.experimental.pallas import tpu as pltpu

N_DEV = 16
M_BLK = 512
K_BLK = 512
N_HALF = 2048


def kernel(x, w_mat):
    m_total, k_shard = x.shape
    k_total, n = w_mat.shape
    assert k_shard == K_BLK and m_total == N_DEV * M_BLK
    n_halves = n // N_HALF
    n_steps = N_DEV * n_halves

    def body(x_ref, w_ref, out_ref, staging, x_send, a2a_buf, w_buf,
             send_sems, recv_sems, st_sems, w_sems):
        me = lax.axis_index("i")

        def w_copy(step, slot):
            d, h = divmod(step, n_halves)
            s = lax.rem(me + (N_DEV - d), N_DEV)
            return pltpu.make_async_copy(
                w_ref.at[pl.ds(s * K_BLK, K_BLK), pl.ds(h * N_HALF, N_HALF)],
                w_buf.at[slot], w_sems.at[slot])

        w_copy(0, 0).start()

        barrier_sem = pltpu.get_barrier_semaphore()
        for d in range(1, N_DEV):
            t = lax.rem(me + d, N_DEV)
            pl.semaphore_signal(
                barrier_sem, inc=1,
                device_id=(t,), device_id_type=pl.DeviceIdType.MESH)
        pl.semaphore_wait(barrier_sem, N_DEV - 1)

        def stage_in(j, slot):
            t = lax.rem(me + j, N_DEV)
            return pltpu.make_async_copy(
                x_ref.at[pl.ds(t * M_BLK, M_BLK), :],
                staging.at[slot], st_sems.at[slot])

        seq = list(range(1, N_DEV)) + [0]
        n_stage = 4
        for k in range(n_stage - 1):
            stage_in(seq[k], k).start()
        rdmas = []
        for idx, j in enumerate(seq):
            slot = idx % n_stage
            if idx + n_stage - 1 < len(seq):
                stage_in(seq[idx + n_stage - 1], (idx + n_stage - 1) % n_stage).start()
            stage_in(j, slot).wait()
            x_send[j] = staging[slot].astype(jnp.bfloat16)
            if j != 0:
                t = lax.rem(me + j, N_DEV)
                rdma = pltpu.make_async_remote_copy(
                    src_ref=x_send.at[j],
                    dst_ref=a2a_buf.at[j],
                    send_sem=send_sems.at[j],
                    recv_sem=recv_sems.at[j],
                    device_id=(t,),
                    device_id_type=pl.DeviceIdType.MESH,
                )
                rdma.start()
                rdmas.append((j, rdma))
        rdma_by_offset = dict(rdmas)

        for d in range(N_DEV):
            for h in range(n_halves):
                step = d * n_halves + h
                if step + 1 < n_steps:
                    w_copy(step + 1, (step + 1) % 2).start()
                w_copy(step, step % 2).wait()
                if h == 0 and d > 0:
                    rdma_by_offset[d].wait_recv()
                lhs = (x_send[0] if d == 0 else a2a_buf[d]).astype(jnp.float32)
                prod = lax.dot_general(
                    lhs, w_buf[step % 2],
                    (((1,), (0,)), ((), ())),
                    preferred_element_type=jnp.float32,
                )
                cols = pl.ds(h * N_HALF, N_HALF)
                if d == 0:
                    out_ref[:, cols] = prod
                else:
                    out_ref[:, cols] += prod

        out_ref[...] = jnp.maximum(out_ref[...], 0.0)

        for _, r in rdmas:
            r.wait_send()

    return pl.pallas_call(
        body,
        out_shape=jax.ShapeDtypeStruct((M_BLK, n), jnp.float32),
        in_specs=[
            pl.BlockSpec(memory_space=pl.ANY),
            pl.BlockSpec(memory_space=pl.ANY),
        ],
        out_specs=pl.BlockSpec(memory_space=pltpu.MemorySpace.VMEM),
        scratch_shapes=[
            pltpu.VMEM((4, M_BLK, K_BLK), jnp.float32),
            pltpu.VMEM((N_DEV, M_BLK, K_BLK), jnp.bfloat16),
            pltpu.VMEM((N_DEV, M_BLK, K_BLK), jnp.bfloat16),
            pltpu.VMEM((2, K_BLK, N_HALF), jnp.float32),
            pltpu.SemaphoreType.DMA((N_DEV,)),
            pltpu.SemaphoreType.DMA((N_DEV,)),
            pltpu.SemaphoreType.DMA((4,)),
            pltpu.SemaphoreType.DMA((2,)),
        ],
        compiler_params=pltpu.CompilerParams(collective_id=0),
    )(x, w_mat)


# device time: 115988 ns/iter; 1.2262x vs baseline; 1.0003x over previous
import jax
import jax.numpy as jnp
from jax import lax
from jax.experimental import pallas as pl
from jax.experimental.pallas import tpu as pltpu

N_DEV = 16
M_BLK = 512
K_BLK = 512
N_HALF = 2048


def kernel(x, w_mat):
    m_total, k_shard = x.shape
    k_total, n = w_mat.shape
    assert k_shard == K_BLK and m_total == N_DEV * M_BLK
    n_halves = n // N_HALF
    n_steps = N_DEV * n_halves

    def body(x_ref, w_ref, out_ref, staging, x_send, a2a_buf, w_buf,
             send_sems, recv_sems, st_sems, w_sems):
        me = lax.axis_index("i")

        def w_copy(step, slot):
            d, h = divmod(step, n_halves)
            s = lax.rem(me + (N_DEV - d), N_DEV)
            return pltpu.make_async_copy(
                w_ref.at[pl.ds(s * K_BLK, K_BLK), pl.ds(h * N_HALF, N_HALF)],
                w_buf.at[slot], w_sems.at[slot])

        w_copy(0, 0).start()

        barrier_sem = pltpu.get_barrier_semaphore()
        for d in range(1, N_DEV):
            t = lax.rem(me + d, N_DEV)
            pl.semaphore_signal(
                barrier_sem, inc=1,
                device_id=(t,), device_id_type=pl.DeviceIdType.MESH)
        pl.semaphore_wait(barrier_sem, N_DEV - 1)

        def stage_in(j, slot):
            t = lax.rem(me + j, N_DEV)
            return pltpu.make_async_copy(
                x_ref.at[pl.ds(t * M_BLK, M_BLK), :],
                staging.at[slot], st_sems.at[slot])

        seq = list(range(1, N_DEV)) + [0]
        n_stage = 4
        for k in range(n_stage - 1):
            stage_in(seq[k], k).start()
        rdmas = []
        for idx, j in enumerate(seq):
            slot = idx % n_stage
            if idx + n_stage - 1 < len(seq):
                stage_in(seq[idx + n_stage - 1], (idx + n_stage - 1) % n_stage).start()
            stage_in(j, slot).wait()
            x_send[j] = staging[slot].astype(jnp.bfloat16)
            if j != 0:
                t = lax.rem(me + j, N_DEV)
                rdma = pltpu.make_async_remote_copy(
                    src_ref=x_send.at[j],
                    dst_ref=a2a_buf.at[j],
                    send_sem=send_sems.at[j],
                    recv_sem=recv_sems.at[j],
                    device_id=(t,),
                    device_id_type=pl.DeviceIdType.MESH,
                )
                rdma.start()
                rdmas.append((j, rdma))
        rdma_by_offset = dict(rdmas)

        for d in range(N_DEV):
            for h in range(n_halves):
                step = d * n_halves + h
                if step + 1 < n_steps:
                    w_copy(step + 1, (step + 1) % 2).start()
                w_copy(step, step % 2).wait()
                if h == 0 and d > 0:
                    rdma_by_offset[d].wait_recv()
                lhs = (x_send[0] if d == 0 else a2a_buf[d]).astype(jnp.float32)
                prod = lax.dot_general(
                    lhs, w_buf[step % 2],
                    (((1,), (0,)), ((), ())),
                    preferred_element_type=jnp.float32,
                )
                cols = pl.ds(h * N_HALF, N_HALF)
                if d == 0:
                    out_ref[:, cols] = prod
                elif d == N_DEV - 1:
                    out_ref[:, cols] = jnp.maximum(out_ref[:, cols] + prod, 0.0)
                else:
                    out_ref[:, cols] += prod

        for _, r in rdmas:
            r.wait_send()

    return pl.pallas_call(
        body,
        out_shape=jax.ShapeDtypeStruct((M_BLK, n), jnp.float32),
        in_specs=[
            pl.BlockSpec(memory_space=pl.ANY),
            pl.BlockSpec(memory_space=pl.ANY),
        ],
        out_specs=pl.BlockSpec(memory_space=pltpu.MemorySpace.VMEM),
        scratch_shapes=[
            pltpu.VMEM((4, M_BLK, K_BLK), jnp.float32),
            pltpu.VMEM((N_DEV, M_BLK, K_BLK), jnp.bfloat16),
            pltpu.VMEM((N_DEV, M_BLK, K_BLK), jnp.bfloat16),
            pltpu.VMEM((2, K_BLK, N_HALF), jnp.float32),
            pltpu.SemaphoreType.DMA((N_DEV,)),
            pltpu.SemaphoreType.DMA((N_DEV,)),
            pltpu.SemaphoreType.DMA((4,)),
            pltpu.SemaphoreType.DMA((2,)),
        ],
        compiler_params=pltpu.CompilerParams(collective_id=0),
    )(x, w_mat)
